# baseline (speedup 1.0000x reference)
import functools

import jax
import jax.numpy as jnp
from jax import lax
from jax.experimental import pallas as pl
from jax.experimental.pallas import tpu as pltpu
from jax.experimental.pallas import tpu_sc as plsc

NUM_CORES = 2
NUM_SUBCORES = 16
NUM_WORKERS = NUM_CORES * NUM_SUBCORES
LANES = 16

B = 16384
C = 5
D = 64
CHUNK = 16
PAIRS = CHUNK * C
B_PER_W = B // NUM_WORKERS
NCHUNKS = B_PER_W // CHUNK
DV = D // LANES


_GDN = lax.GatherDimensionNumbers(
    offset_dims=(), collapsed_slice_dims=(0,), start_index_map=(0,))


def _perm(x, idx):
    return lax.gather(x, idx[:, None], _GDN, (1,),
                      mode=lax.GatherScatterMode.PROMISE_IN_BOUNDS)


def _sc_body(tgt_idx_hbm, ctx_idx_hbm, tgt_table, ctx_table, out_hbm,
             idx_t_v, idx_c_v, tgt_rows, ctx_rows, out_v, sem_t, sem_c):
    wid = lax.axis_index("s") * NUM_CORES + lax.axis_index("c")
    lane_iota = lax.iota(jnp.int32, LANES)
    lane_masks = [lane_iota == l for l in range(LANES)]
    xor_idx = [jnp.bitwise_xor(lane_iota, s) for s in (8, 4, 2, 1)]

    def chunk_body(ch, carry):
        base = wid * B_PER_W + ch * CHUNK
        pltpu.sync_copy(tgt_idx_hbm.at[pl.ds(base, CHUNK)], idx_t_v)
        pltpu.sync_copy(ctx_idx_hbm.at[pl.ds(base * C, PAIRS)], idx_c_v)
        cp_t = pltpu.async_copy(tgt_table.at[idx_t_v], tgt_rows, sem_t)
        cp_c = pltpu.async_copy(ctx_table.at[idx_c_v], ctx_rows, sem_c)
        cp_t.wait()
        cp_c.wait()

        acc = [jnp.zeros((LANES,), jnp.float32) for _ in range(C)]
        for ii in range(CHUNK):
            t = [tgt_rows[ii, pl.ds(k * LANES, LANES)] for k in range(DV)]
            for c in range(C):
                r = ii * C + c
                cv = [ctx_rows[r, pl.ds(k * LANES, LANES)] for k in range(DV)]
                s = (cv[0] * t[0] + cv[1] * t[1]) + (cv[2] * t[2] + cv[3] * t[3])
                for xi in xor_idx:
                    s = s + _perm(s, xi)
                acc[r // LANES] = jnp.where(lane_masks[r % LANES], s, acc[r // LANES])
        for v in range(C):
            out_v[pl.ds(v * LANES, LANES)] = acc[v]
        pltpu.sync_copy(out_v, out_hbm.at[pl.ds(base * C, PAIRS)])
        return carry

    lax.fori_loop(0, NCHUNKS, chunk_body, 0)


@jax.jit
def _sc_call(tgt_idx, ctx_idx, tgt_table, ctx_table):
    mesh = plsc.VectorSubcoreMesh(core_axis_name="c", subcore_axis_name="s")
    return pl.kernel(
        _sc_body,
        out_type=jax.ShapeDtypeStruct((B * C,), jnp.float32),
        mesh=mesh,
        compiler_params=pltpu.CompilerParams(use_tc_tiling_on_sc=False),
        scratch_types=[
            pltpu.VMEM((CHUNK,), jnp.int32),
            pltpu.VMEM((PAIRS,), jnp.int32),
            pltpu.VMEM((CHUNK, D), jnp.float32),
            pltpu.VMEM((PAIRS, D), jnp.float32),
            pltpu.VMEM((PAIRS,), jnp.float32),
            pltpu.SemaphoreType.DMA,
            pltpu.SemaphoreType.DMA,
        ],
    )(tgt_idx, ctx_idx, tgt_table, ctx_table)


def kernel(target, context, target_table, context_table):
    tgt_idx = target.reshape(B)
    ctx_idx = context.reshape(B * C)
    out = _sc_call(tgt_idx, ctx_idx, target_table, context_table)
    return out.reshape(B, C)
